# baseline (speedup 1.0000x reference)
import jax
import jax.numpy as jnp
from jax import lax
from jax.experimental import pallas as pl
from jax.experimental.pallas import tpu as pltpu
from jax.experimental.pallas import tpu_sc as plsc

_BATCH, _CH, _LENGTH = 4, 1024, 4096
_CB = 128
_NCB = _CH // _CB
_L_QUARTER = _LENGTH // 4
_L_CHUNK = 256
_N_LH = _L_QUARTER // _L_CHUNK
_C_STEP = 8
_N_ST = _CB // _C_STEP
_N_LV = _L_CHUNK // 16


def _sc_body(x_hbm, emb_hbm, out_hbm, pos_buf, xb, emb_sems, in_sems, out_sems):
    cid = lax.axis_index("c")
    sid = lax.axis_index("s")
    wid = sid * 2 + cid
    cb = wid % _NCB
    lq = wid // _NCB
    c1 = cb * _CB
    lq0 = lq * _L_QUARTER
    lane = lax.iota(jnp.int32, 16)

    def emb_src(lh):
        return emb_hbm.at[pl.ds(lq0 + lh * _L_CHUNK, _L_CHUNK), pl.ds(c1, _CB)]

    def x_slice(hbm, lh, st):
        return hbm.at[:, pl.ds(c1 + st * _C_STEP, _C_STEP),
                      pl.ds(lq0 + lh * _L_CHUNK, _L_CHUNK)]

    pltpu.async_copy(emb_src(0), pos_buf.at[0], emb_sems.at[0])

    def lh_body(lh, carry):
        pp = lax.rem(lh, 2)
        pltpu.make_async_copy(emb_src(lh), pos_buf.at[pp],
                              emb_sems.at[pp]).wait()

        @pl.when(lh + 1 < _N_LH)
        def _():
            pltpu.async_copy(emb_src(lh + 1), pos_buf.at[1 - pp],
                             emb_sems.at[1 - pp])

        pltpu.async_copy(x_slice(x_hbm, lh, 0), xb.at[0], in_sems.at[0])
        pltpu.async_copy(x_slice(x_hbm, lh, 1), xb.at[1], in_sems.at[1])
        pltpu.async_copy(x_slice(x_hbm, lh, 2), xb.at[2], in_sems.at[2])

        def st_body(st, carry2):
            s = lax.rem(st, 4)
            pltpu.make_async_copy(x_slice(x_hbm, lh, st), xb.at[s],
                                  in_sems.at[s]).wait()

            @plsc.parallel_loop(0, _C_STEP * _N_LV, unroll=8)
            def _(i):
                cr = i // _N_LV
                lv = i % _N_LV
                cvec = jnp.full((16,), 0, jnp.int32) + (st * _C_STEP + cr)
                sl = pl.ds(lv * 16, 16)
                pos_vec = plsc.load_gather(pos_buf.at[pp],
                                           [lv * 16 + lane, cvec])
                for b in range(_BATCH):
                    plsc.addupdate(xb.at[s, b, cr, sl], pos_vec)

            pltpu.async_copy(xb.at[s], x_slice(out_hbm, lh, st), out_sems.at[s])

            @pl.when(st + 3 < _N_ST)
            def _():
                s2 = lax.rem(st + 3, 4)

                @pl.when(st >= 1)
                def _():
                    pltpu.make_async_copy(xb.at[s2],
                                          x_slice(out_hbm, lh, st - 1),
                                          out_sems.at[s2]).wait()

                pltpu.async_copy(x_slice(x_hbm, lh, st + 3), xb.at[s2],
                                 in_sems.at[s2])

            return carry2

        lax.fori_loop(0, _N_ST, st_body, 0)

        for tail in range(4):
            st = _N_ST - 4 + tail
            s = lax.rem(st, 4)
            pltpu.make_async_copy(xb.at[s], x_slice(out_hbm, lh, st),
                                  out_sems.at[s]).wait()
        return carry

    lax.fori_loop(0, _N_LH, lh_body, 0)


@jax.jit
def _pos_add(x, emb):
    mesh = plsc.VectorSubcoreMesh(core_axis_name="c", subcore_axis_name="s")
    return pl.kernel(
        _sc_body,
        out_type=jax.ShapeDtypeStruct((_BATCH, _CH, _LENGTH), jnp.float32),
        mesh=mesh,
        scratch_types=[
            pltpu.VMEM((2, _L_CHUNK, _CB), jnp.float32),
            pltpu.VMEM((4, _BATCH, _C_STEP, _L_CHUNK), jnp.float32),
            pltpu.SemaphoreType.DMA((2,)),
            pltpu.SemaphoreType.DMA((4,)),
            pltpu.SemaphoreType.DMA((4,)),
        ],
        compiler_params=pltpu.CompilerParams(needs_layout_passes=False),
    )(x, emb)


def kernel(x, emb):
    return _pos_add(x, emb)


# speedup vs baseline: 2.6923x; 2.6923x over previous
import jax
import jax.numpy as jnp
from jax.experimental import pallas as pl

_CH_BLK = 128
_LEN_BLK = 4096


def _add_pos_kernel(x_ref, emb_ref, out_ref):
    pos = jnp.transpose(emb_ref[...], (1, 0))
    out_ref[...] = x_ref[...] + pos[None, :, :]


def kernel(x, emb):
    batch, ch, length = x.shape
    grid = (ch // _CH_BLK, length // _LEN_BLK)
    return pl.pallas_call(
        _add_pos_kernel,
        grid=grid,
        in_specs=[
            pl.BlockSpec((batch, _CH_BLK, _LEN_BLK), lambda i, j: (0, i, j)),
            pl.BlockSpec((_LEN_BLK, _CH_BLK), lambda i, j: (j, i)),
        ],
        out_specs=pl.BlockSpec((batch, _CH_BLK, _LEN_BLK), lambda i, j: (0, i, j)),
        out_shape=jax.ShapeDtypeStruct(x.shape, x.dtype),
    )(x, emb)
